# baseline (speedup 1.0000x reference)
import functools

import jax
import jax.numpy as jnp
from jax import lax
from jax.experimental import pallas as pl
from jax.experimental.pallas import tpu as pltpu
from jax.experimental.pallas import tpu_sc as plsc

NC = 2
NS = 16
NW = NC * NS
LANES = 16


def _stage_offsets(ps_hbm, ps_v, ps_sm, B, T):
    nmem = min(LANES, B)
    pltpu.sync_copy(ps_hbm.at[pl.ds(0, nmem)], ps_v.at[pl.ds(0, nmem)])
    for k0 in range(0, B, LANES):
        v = ps_v[pl.ds(k0, LANES)]
        for k in range(min(LANES, B - k0)):
            ps_sm[k0 + k] = v[k]
    ps_sm[B] = jnp.int32(T)


def _make_phase1(T, D, B, CH):
    NJ = D // LANES
    RW = T // NW
    NCHUNK = RW // CH
    NP = NCHUNK // 2
    assert NCHUNK % 2 == 0
    mesh = plsc.VectorSubcoreMesh(core_axis_name="c", subcore_axis_name="s")

    @functools.partial(
        pl.kernel,
        out_type=(
            jax.ShapeDtypeStruct((NW, B, D), jnp.float32),
            jax.ShapeDtypeStruct((NW, B, D), jnp.float32),
        ),
        mesh=mesh,
        scratch_types=[
            pltpu.VMEM((CH, D), jnp.float32),
            pltpu.VMEM((CH, D), jnp.float32),
            pltpu.VMEM((B, D), jnp.float32),
            pltpu.VMEM((B, D), jnp.float32),
            pltpu.VMEM((LANES,), jnp.int32),
            pltpu.SMEM((32,), jnp.int32),
            pltpu.SemaphoreType.DMA,
            pltpu.SemaphoreType.DMA,
        ],
    )
    def phase1(x_hbm, ps_hbm, pmax_hbm, psum_hbm, bufa, bufb, m_v, s_v, ps_v,
               ps_sm, isem_a, isem_b):
        wid = lax.axis_index("s") * NC + lax.axis_index("c")
        lo = wid * RW

        pltpu.async_copy(x_hbm.at[pl.ds(lo, CH), :], bufa, isem_a)
        if NCHUNK > 1:
            pltpu.async_copy(x_hbm.at[pl.ds(lo + CH, CH), :], bufb, isem_b)

        _stage_offsets(ps_hbm, ps_v, ps_sm, B, T)

        neg = jnp.full((LANES,), -jnp.inf, jnp.float32)
        zero = jnp.zeros((LANES,), jnp.float32)

        def init_body(sb, carry):
            for j in range(NJ):
                m_v[sb, pl.ds(j * LANES, LANES)] = neg
                s_v[sb, pl.ds(j * LANES, LANES)] = zero
            return carry

        lax.fori_loop(0, B, init_body, 0)

        def process(buf, base):
            def seg_body(sb, carry):
                r0 = jnp.clip(ps_sm[sb] - base, 0, CH)
                r1 = jnp.clip(ps_sm[sb + 1] - base, 0, CH)

                @pl.when(r1 > r0)
                def _():
                    @plsc.parallel_loop(r0, r1, unroll=4, carry=(neg,) * NJ)
                    def cmax(r, acc):
                        return tuple(
                            jnp.maximum(acc[j], buf[r, pl.ds(j * LANES, LANES)])
                            for j in range(NJ)
                        )

                    mnew = []
                    for j in range(NJ):
                        sl = pl.ds(j * LANES, LANES)
                        mo = m_v[sb, sl]
                        mn = jnp.maximum(mo, cmax[j])
                        s_v[sb, sl] = s_v[sb, sl] * jnp.exp(mo - mn)
                        m_v[sb, sl] = mn
                        mnew.append(mn)

                    @plsc.parallel_loop(r0, r1, unroll=4, carry=(zero,) * NJ)
                    def ssum(r, acc):
                        return tuple(
                            acc[j]
                            + jnp.exp(buf[r, pl.ds(j * LANES, LANES)] - mnew[j])
                            for j in range(NJ)
                        )

                    for j in range(NJ):
                        sl = pl.ds(j * LANES, LANES)
                        s_v[sb, sl] = s_v[sb, sl] + ssum[j]

                return carry

            lax.fori_loop(0, B, seg_body, 0)

        def pair_body(p, carry):
            base0 = lo + (2 * p) * CH
            pltpu.make_async_copy(
                x_hbm.at[pl.ds(base0, CH), :], bufa, isem_a).wait()
            process(bufa, base0)

            @pl.when(p + 1 < NP)
            def _():
                pltpu.async_copy(
                    x_hbm.at[pl.ds(base0 + 2 * CH, CH), :], bufa, isem_a)

            pltpu.make_async_copy(
                x_hbm.at[pl.ds(base0 + CH, CH), :], bufb, isem_b).wait()
            process(bufb, base0 + CH)

            @pl.when(p + 1 < NP)
            def _():
                pltpu.async_copy(
                    x_hbm.at[pl.ds(base0 + 3 * CH, CH), :], bufb, isem_b)

            return carry

        lax.fori_loop(0, NP, pair_body, 0)

        pltpu.sync_copy(m_v, pmax_hbm.at[wid])
        pltpu.sync_copy(s_v, psum_hbm.at[wid])

    return phase1


def _ln(z):
    bits = lax.bitcast_convert_type(z, jnp.int32)
    eb = bits >> 23
    m = lax.bitcast_convert_type(bits - (eb << 23) + (127 << 23), jnp.float32)
    e = eb - 127
    big = m > 1.4142135623730951
    m = jnp.where(big, m * 0.5, m)
    e = e + jnp.where(big, 1, 0)
    s = (m - 1.0) / (m + 1.0)
    s2 = s * s
    p = 1.0 + s2 * (1 / 3 + s2 * (1 / 5 + s2 * (1 / 7 + s2 * (1 / 9))))
    return e.astype(jnp.float32) * 0.6931471805599453 + 2.0 * s * p


def _make_phase23(T, D, B, CH):
    NJ = D // LANES
    RW = T // NW
    NCHUNK = RW // CH
    NP = NCHUNK // 2
    assert NCHUNK % 2 == 0
    assert B == NS
    mesh = plsc.VectorSubcoreMesh(core_axis_name="c", subcore_axis_name="s")

    @functools.partial(
        pl.kernel,
        out_type=jax.ShapeDtypeStruct((T, D), jnp.float32),
        mesh=mesh,
        scratch_types=[
            pltpu.VMEM((CH, D), jnp.float32),
            pltpu.VMEM((CH, D), jnp.float32),
            pltpu.VMEM((B, D), jnp.float32),
            pltpu.VMEM((NW, D), jnp.float32),
            pltpu.VMEM((NW, D), jnp.float32),
            pltpu.VMEM((1, D), jnp.float32),
            pltpu.VMEM_SHARED((B, D), jnp.float32),
            pltpu.VMEM((LANES,), jnp.int32),
            pltpu.SMEM((32,), jnp.int32),
            pltpu.SemaphoreType.DMA,
            pltpu.SemaphoreType.DMA,
            pltpu.SemaphoreType.DMA,
            pltpu.SemaphoreType.DMA,
        ],
    )
    def phase23(x_hbm, ps_hbm, pmax_hbm, psum_hbm, out_hbm, bufa, bufb, b_v,
                pmx_v, psm_v, bseg_v, b_sh, ps_v, ps_sm,
                isem_a, isem_b, osem_a, osem_b):
        cid = lax.axis_index("c")
        sid = lax.axis_index("s")
        wid = sid * NC + cid
        lo = wid * RW

        pltpu.async_copy(x_hbm.at[pl.ds(lo, CH), :], bufa, isem_a)
        if NCHUNK > 1:
            pltpu.async_copy(x_hbm.at[pl.ds(lo + CH, CH), :], bufb, isem_b)

        _stage_offsets(ps_hbm, ps_v, ps_sm, B, T)

        pltpu.sync_copy(pmax_hbm.at[:, sid, :], pmx_v)
        pltpu.sync_copy(psum_hbm.at[:, sid, :], psm_v)

        neg = jnp.full((LANES,), -jnp.inf, jnp.float32)
        zero = jnp.zeros((LANES,), jnp.float32)

        def mx_body(w, acc):
            return tuple(
                jnp.maximum(acc[j], pmx_v[w, pl.ds(j * LANES, LANES)])
                for j in range(NJ)
            )

        mseg = lax.fori_loop(0, NW, mx_body, (neg,) * NJ)

        def z_body(w, acc):
            out = []
            for j in range(NJ):
                sl = pl.ds(j * LANES, LANES)
                sp = psm_v[w, sl]
                out.append(
                    acc[j]
                    + jnp.where(sp > 0, sp * jnp.exp(pmx_v[w, sl] - mseg[j]),
                                zero)
                )
            return tuple(out)

        zseg = lax.fori_loop(0, NW, z_body, (zero,) * NJ)

        for j in range(NJ):
            bseg_v[0, pl.ds(j * LANES, LANES)] = mseg[j] + _ln(zseg[j])
        pltpu.sync_copy(bseg_v, b_sh.at[pl.ds(sid, 1), :])
        plsc.subcore_barrier()
        pltpu.sync_copy(b_sh, b_v)

        def process(buf, base):
            def seg_body(sb, carry):
                r0 = jnp.clip(ps_sm[sb] - base, 0, CH)
                r1 = jnp.clip(ps_sm[sb + 1] - base, 0, CH)

                @pl.when(r1 > r0)
                def _():
                    bj = [b_v[sb, pl.ds(j * LANES, LANES)] for j in range(NJ)]

                    @plsc.parallel_loop(r0, r1, unroll=4)
                    def _sub(r):
                        for j in range(NJ):
                            sl = pl.ds(j * LANES, LANES)
                            buf[r, sl] = buf[r, sl] - bj[j]

                return carry

            lax.fori_loop(0, B, seg_body, 0)

        def pair_body(p, carry):
            base0 = lo + (2 * p) * CH
            base1 = base0 + CH
            pltpu.make_async_copy(
                x_hbm.at[pl.ds(base0, CH), :], bufa, isem_a).wait()
            process(bufa, base0)
            pltpu.async_copy(bufa, out_hbm.at[pl.ds(base0, CH), :], osem_a)

            pltpu.make_async_copy(
                x_hbm.at[pl.ds(base1, CH), :], bufb, isem_b).wait()
            process(bufb, base1)
            pltpu.async_copy(bufb, out_hbm.at[pl.ds(base1, CH), :], osem_b)

            @pl.when(p + 1 < NP)
            def _():
                pltpu.make_async_copy(
                    bufa, out_hbm.at[pl.ds(base0, CH), :], osem_a).wait()
                pltpu.async_copy(
                    x_hbm.at[pl.ds(base0 + 2 * CH, CH), :], bufa, isem_a)
                pltpu.make_async_copy(
                    bufb, out_hbm.at[pl.ds(base1, CH), :], osem_b).wait()
                pltpu.async_copy(
                    x_hbm.at[pl.ds(base1 + 2 * CH, CH), :], bufb, isem_b)

            return carry

        lax.fori_loop(0, NP, pair_body, 0)

        last0 = lo + (NCHUNK - 2) * CH
        pltpu.make_async_copy(
            bufa, out_hbm.at[pl.ds(last0, CH), :], osem_a).wait()
        pltpu.make_async_copy(
            bufb, out_hbm.at[pl.ds(last0 + CH, CH), :], osem_b).wait()

    return phase23


def kernel(logits, prefix_sum):
    T, D = logits.shape
    B = prefix_sum.shape[0] - 1
    CH = 256

    pm, psm = _make_phase1(T, D, B, CH)(logits, prefix_sum)
    return _make_phase23(T, D, B, CH)(logits, prefix_sum, pm, psm)


# speedup vs baseline: 1.0117x; 1.0117x over previous
import functools

import jax
import jax.numpy as jnp
from jax import lax
from jax.experimental import pallas as pl
from jax.experimental.pallas import tpu as pltpu
from jax.experimental.pallas import tpu_sc as plsc

NC = 2
NS = 16
LANES = 16

CH = 128
RC = 13
PC = 3


def _stage_offsets(ps_hbm, ps_v, ps_sm, B, T):
    nmem = min(LANES, B)
    pltpu.sync_copy(ps_hbm.at[pl.ds(0, nmem)], ps_v.at[pl.ds(0, nmem)])
    for k0 in range(0, B, LANES):
        v = ps_v[pl.ds(k0, LANES)]
        for k in range(min(LANES, B - k0)):
            ps_sm[k0 + k] = v[k]
    ps_sm[B] = jnp.int32(T)


def _ln(z):
    bits = lax.bitcast_convert_type(z, jnp.int32)
    eb = bits >> 23
    m = lax.bitcast_convert_type(bits - (eb << 23) + (127 << 23), jnp.float32)
    e = eb - 127
    big = m > 1.4142135623730951
    m = jnp.where(big, m * 0.5, m)
    e = e + jnp.where(big, 1, 0)
    s = (m - 1.0) / (m + 1.0)
    s2 = s * s
    p = 1.0 + s2 * (1 / 3 + s2 * (1 / 5 + s2 * (1 / 7 + s2 * (1 / 9))))
    return e.astype(jnp.float32) * 0.6931471805599453 + 2.0 * s * p


def _make_fused(T, D, B):
    COLS = D // NC
    NJ = COLS // LANES
    RW = T // NS
    assert RW == (RC + PC) * CH
    assert B == NS
    SLAB = RC * CH
    mesh = plsc.VectorSubcoreMesh(core_axis_name="c", subcore_axis_name="s")

    @functools.partial(
        pl.kernel,
        out_type=(
            jax.ShapeDtypeStruct((T, D), jnp.float32),
            jax.ShapeDtypeStruct((NC, NS, B, COLS), jnp.float32),
            jax.ShapeDtypeStruct((NC, NS, B, COLS), jnp.float32),
        ),
        compiler_params=pltpu.CompilerParams(use_tc_tiling_on_sc=False),
        mesh=mesh,
        scratch_types=[
            pltpu.VMEM((SLAB, COLS), jnp.float32),
            pltpu.VMEM((CH, COLS), jnp.float32),
            pltpu.VMEM((CH, COLS), jnp.float32),
            pltpu.VMEM((B, COLS), jnp.float32),
            pltpu.VMEM((B, COLS), jnp.float32),
            pltpu.VMEM((B, COLS), jnp.float32),
            pltpu.VMEM((1, COLS), jnp.float32),
            pltpu.VMEM((LANES,), jnp.int32),
            pltpu.SMEM((32,), jnp.int32),
            pltpu.VMEM_SHARED((B, COLS), jnp.float32),
            pltpu.SemaphoreType.DMA,
            pltpu.SemaphoreType.DMA,
            pltpu.SemaphoreType.DMA,
            pltpu.SemaphoreType.DMA,
            pltpu.SemaphoreType.DMA,
            pltpu.SemaphoreType.DMA,
        ],
    )
    def fused(x_hbm, ps_hbm, out_hbm, pm_hbm, psm_hbm, slab, bnca, bncb,
              m_v, s_v, b_v, bseg_v, ps_v, ps_sm, b_sh,
              ssem, isem_a, isem_b, oslab, osem_a, osem_b):
        cid = lax.axis_index("c")
        sid = lax.axis_index("s")
        c0 = cid * COLS
        r_lo = sid * RW

        _stage_offsets(ps_hbm, ps_v, ps_sm, B, T)

        for ci in range(RC):
            pltpu.async_copy(
                x_hbm.at[pl.ds(r_lo + ci * CH, CH), pl.ds(c0, COLS)],
                slab.at[pl.ds(ci * CH, CH), :], ssem)
        bncs = (bnca, bncb)
        isems = (isem_a, isem_b)
        osems = (osem_a, osem_b)
        for cj in range(min(PC, 2)):
            pltpu.async_copy(
                x_hbm.at[pl.ds(r_lo + (RC + cj) * CH, CH), pl.ds(c0, COLS)],
                bncs[cj], isems[cj])

        neg = jnp.full((LANES,), -jnp.inf, jnp.float32)
        zero = jnp.zeros((LANES,), jnp.float32)

        def init_body(sb, carry):
            for j in range(NJ):
                m_v[sb, pl.ds(j * LANES, LANES)] = neg
                s_v[sb, pl.ds(j * LANES, LANES)] = zero
            return carry

        lax.fori_loop(0, B, init_body, 0)

        def accum(buf, base, lo_r, hi_r):
            def seg_body(sb, carry):
                r0 = jnp.clip(ps_sm[sb] - base, lo_r, hi_r)
                r1 = jnp.clip(ps_sm[sb + 1] - base, lo_r, hi_r)

                @pl.when(r1 > r0)
                def _():
                    @plsc.parallel_loop(r0, r1, unroll=4, carry=(neg,) * NJ)
                    def cmax(r, acc):
                        return tuple(
                            jnp.maximum(acc[j], buf[r, pl.ds(j * LANES, LANES)])
                            for j in range(NJ)
                        )

                    mnew = []
                    for j in range(NJ):
                        sl = pl.ds(j * LANES, LANES)
                        mo = m_v[sb, sl]
                        mn = jnp.maximum(mo, cmax[j])
                        s_v[sb, sl] = s_v[sb, sl] * jnp.exp(mo - mn)
                        m_v[sb, sl] = mn
                        mnew.append(mn)

                    @plsc.parallel_loop(r0, r1, unroll=4, carry=(zero,) * NJ)
                    def ssum(r, acc):
                        return tuple(
                            acc[j]
                            + jnp.exp(buf[r, pl.ds(j * LANES, LANES)] - mnew[j])
                            for j in range(NJ)
                        )

                    for j in range(NJ):
                        sl = pl.ds(j * LANES, LANES)
                        s_v[sb, sl] = s_v[sb, sl] + ssum[j]

                return carry

            lax.fori_loop(0, B, seg_body, 0)

        def res_body(ci, carry):
            pltpu.make_async_copy(
                x_hbm.at[pl.ds(r_lo + ci * CH, CH), pl.ds(c0, COLS)],
                slab.at[pl.ds(ci * CH, CH), :], ssem).wait()
            accum(slab, r_lo, ci * CH, ci * CH + CH)
            return carry

        lax.fori_loop(0, RC, res_body, 0)

        for cj in range(PC):
            base = r_lo + (RC + cj) * CH
            pltpu.make_async_copy(
                x_hbm.at[pl.ds(base, CH), pl.ds(c0, COLS)],
                bncs[cj % 2], isems[cj % 2]).wait()
            accum(bncs[cj % 2], base, 0, CH)
            if cj + 2 < PC:
                nbase = r_lo + (RC + cj + 2) * CH
                pltpu.async_copy(
                    x_hbm.at[pl.ds(nbase, CH), pl.ds(c0, COLS)],
                    bncs[cj % 2], isems[cj % 2])

        pltpu.sync_copy(m_v, pm_hbm.at[cid, sid])
        pltpu.sync_copy(s_v, psm_hbm.at[cid, sid])
        plsc.subcore_barrier()
        pltpu.sync_copy(pm_hbm.at[cid, :, sid, :], bnca.at[pl.ds(0, NS), :])
        pltpu.sync_copy(psm_hbm.at[cid, :, sid, :], bncb.at[pl.ds(0, NS), :])

        def mx_body(w, acc):
            return tuple(
                jnp.maximum(acc[j], bnca[w, pl.ds(j * LANES, LANES)])
                for j in range(NJ)
            )

        mseg = lax.fori_loop(0, NS, mx_body, (neg,) * NJ)

        def z_body(w, acc):
            out = []
            for j in range(NJ):
                sl = pl.ds(j * LANES, LANES)
                sp = bncb[w, sl]
                out.append(
                    acc[j]
                    + jnp.where(sp > 0, sp * jnp.exp(bnca[w, sl] - mseg[j]),
                                zero)
                )
            return tuple(out)

        zseg = lax.fori_loop(0, NS, z_body, (zero,) * NJ)

        for j in range(NJ):
            bseg_v[0, pl.ds(j * LANES, LANES)] = mseg[j] + _ln(zseg[j])
        pltpu.sync_copy(bseg_v, b_sh.at[pl.ds(sid, 1), :])
        plsc.subcore_barrier()
        pltpu.sync_copy(b_sh, b_v)

        for cj in range(min(PC, 2)):
            base = r_lo + (RC + cj) * CH
            pltpu.async_copy(
                x_hbm.at[pl.ds(base, CH), pl.ds(c0, COLS)],
                bncs[cj % 2], isems[cj % 2])

        def subtract(buf, base, lo_r, hi_r):
            def seg_body(sb, carry):
                r0 = jnp.clip(ps_sm[sb] - base, lo_r, hi_r)
                r1 = jnp.clip(ps_sm[sb + 1] - base, lo_r, hi_r)

                @pl.when(r1 > r0)
                def _():
                    bj = [b_v[sb, pl.ds(j * LANES, LANES)] for j in range(NJ)]

                    @plsc.parallel_loop(r0, r1, unroll=4)
                    def _sub(r):
                        for j in range(NJ):
                            sl = pl.ds(j * LANES, LANES)
                            buf[r, sl] = buf[r, sl] - bj[j]

                return carry

            lax.fori_loop(0, B, seg_body, 0)

        def out_body(ci, carry):
            subtract(slab, r_lo, ci * CH, ci * CH + CH)
            pltpu.async_copy(
                slab.at[pl.ds(ci * CH, CH), :],
                out_hbm.at[pl.ds(r_lo + ci * CH, CH), pl.ds(c0, COLS)], oslab)
            return carry

        lax.fori_loop(0, RC, out_body, 0)

        for cj in range(PC):
            base = r_lo + (RC + cj) * CH
            pltpu.make_async_copy(
                x_hbm.at[pl.ds(base, CH), pl.ds(c0, COLS)],
                bncs[cj % 2], isems[cj % 2]).wait()
            subtract(bncs[cj % 2], base, 0, CH)
            pltpu.async_copy(
                bncs[cj % 2],
                out_hbm.at[pl.ds(base, CH), pl.ds(c0, COLS)], osems[cj % 2])
            if cj + 2 < PC:
                pltpu.make_async_copy(
                    bncs[cj % 2],
                    out_hbm.at[pl.ds(base, CH), pl.ds(c0, COLS)],
                    osems[cj % 2]).wait()
                pltpu.async_copy(
                    x_hbm.at[pl.ds(base + 2 * CH, CH), pl.ds(c0, COLS)],
                    bncs[cj % 2], isems[cj % 2])

        def drain_body(ci, carry):
            pltpu.make_async_copy(
                slab.at[pl.ds(ci * CH, CH), :],
                out_hbm.at[pl.ds(r_lo + ci * CH, CH), pl.ds(c0, COLS)],
                oslab).wait()
            return carry

        lax.fori_loop(0, RC, drain_body, 0)
        for cj in range(max(PC - 2, 0), PC):
            base = r_lo + (RC + cj) * CH
            pltpu.make_async_copy(
                bncs[cj % 2],
                out_hbm.at[pl.ds(base, CH), pl.ds(c0, COLS)],
                osems[cj % 2]).wait()

    return fused


def kernel(logits, prefix_sum):
    T, D = logits.shape
    B = prefix_sum.shape[0] - 1
    out, _, _ = _make_fused(T, D, B)(logits, prefix_sum)
    return out
